# baseline (speedup 1.0000x reference)
import functools

import jax
import jax.numpy as jnp
from jax import lax
from jax.experimental import pallas as pl
from jax.experimental.pallas import tpu as pltpu
from jax.experimental.pallas import tpu_sc as plsc

NUM_EMB = 100000
DIM = 64
BATCH = 1024

NUM_CORES = 2
NUM_SUBCORES = 16
L = 16
NCHUNK = 16
NROUND = 4
CW = 6272
LAST_BASE = ((NUM_EMB + 127) // 128) * 128 - CW
DPC = DIM // NUM_CORES
NGRP = DPC // 8
RPT = DPC // NUM_SUBCORES


def _sc_body(i1_hbm, i2_hbm, wt_hbm, out_hbm,
             i1_v, i2_v, scale_v, slab_a, slab_b, part_v,
             fb0, fb1, fb2, fb3, out_v, shared, sem_a, sem_b):
    s = lax.axis_index("s")
    c = lax.axis_index("c")
    glocal = s % NGRP
    owner = s // NGRP
    dbase = DPC * c + 8 * glocal

    pltpu.sync_copy(i1_hbm.at[pl.ds(0, BATCH)], i1_v)
    pltpu.sync_copy(i2_hbm.at[pl.ds(0, BATCH)], i2_v)

    iota = lax.iota(jnp.int32, L)
    slabs = [slab_a, slab_b]
    sems = [sem_a, sem_b]

    def _chunk(r):
        q = owner + NGRP * r
        dma_base = pl.multiple_of(
            jnp.where(q == NCHUNK - 1, LAST_BASE, q * CW), 128)
        own_lo = q * CW
        own_hi = jnp.where(q == NCHUNK - 1, NUM_EMB, (q + 1) * CW)
        return dma_base, own_lo, own_hi

    def _fire(r):
        dma_base, _, _ = _chunk(r)
        return pltpu.async_copy(
            wt_hbm.at[pl.ds(dbase, 8), pl.ds(dma_base, CW)],
            slabs[r % 2], sems[r % 2])

    def _prep(t, carry):
        a = i1_v[pl.ds(t * L, L)]
        b = i2_v[pl.ds(t * L, L)]
        scale_v[pl.ds(t * L, L)] = jnp.where(
            a != b, jnp.float32(1.0), jnp.float32(0.0))
        z = jnp.zeros((L,), jnp.float32)
        for dl in range(8):
            part_v[dl, pl.ds(t * L, L)] = z
        return carry

    def _scan(r):
        slab = slabs[r % 2]
        dma_base, own_lo, own_hi = _chunk(r)

        def _accum(t, carry, idx_ref, scaled):
            idx = idx_ref[pl.ds(t * L, L)]
            m = (idx >= own_lo) & (idx < own_hi)
            local = idx - dma_base
            bvec = t * L + iota
            if scaled:
                scl = scale_v[pl.ds(t * L, L)]
            for dl in range(8):
                dsplat = jnp.full((L,), dl, jnp.int32)
                v = plsc.load_gather(slab, [dsplat, local], mask=m)
                if scaled:
                    v = v * scl
                plsc.addupdate_scatter(part_v, [dsplat, bvec], v, mask=m)
            return carry

        lax.fori_loop(0, BATCH // L,
                      functools.partial(_accum, idx_ref=i1_v, scaled=False), 0)
        lax.fori_loop(0, BATCH // L,
                      functools.partial(_accum, idx_ref=i2_v, scaled=True), 0)

    cp = _fire(0)
    lax.fori_loop(0, BATCH // L, _prep, 0)
    cp.wait()
    for r in range(NROUND):
        if r + 1 < NROUND:
            nxt = _fire(r + 1)
        _scan(r)
        if r + 1 < NROUND:
            nxt.wait()

    pltpu.sync_copy(part_v, shared.at[owner, pl.ds(8 * glocal, 8), :])
    plsc.subcore_barrier()

    rbase = RPT * s
    pltpu.sync_copy(shared.at[0, pl.ds(rbase, RPT), :], fb0)
    pltpu.sync_copy(shared.at[1, pl.ds(rbase, RPT), :], fb1)
    pltpu.sync_copy(shared.at[2, pl.ds(rbase, RPT), :], fb2)
    pltpu.sync_copy(shared.at[3, pl.ds(rbase, RPT), :], fb3)

    def _reduce(t, carry):
        for row in range(RPT):
            sl = pl.ds(t * L, L)
            out_v[row, sl] = ((fb0[row, sl] + fb1[row, sl])
                              + (fb2[row, sl] + fb3[row, sl]))
        return carry
    lax.fori_loop(0, BATCH // L, _reduce, 0)

    pltpu.sync_copy(out_v, out_hbm.at[pl.ds(DPC * c + rbase, RPT), :])


_two_hot_sc = functools.partial(
    pl.kernel,
    out_type=jax.ShapeDtypeStruct((DIM, BATCH), jnp.float32),
    mesh=plsc.VectorSubcoreMesh(core_axis_name="c", subcore_axis_name="s"),
    compiler_params=pltpu.CompilerParams(needs_layout_passes=False),
    scratch_types=[
        pltpu.VMEM((BATCH,), jnp.int32),
        pltpu.VMEM((BATCH,), jnp.int32),
        pltpu.VMEM((BATCH,), jnp.float32),
        pltpu.VMEM((8, CW), jnp.float32),
        pltpu.VMEM((8, CW), jnp.float32),
        pltpu.VMEM((8, BATCH), jnp.float32),
        pltpu.VMEM((RPT, BATCH), jnp.float32),
        pltpu.VMEM((RPT, BATCH), jnp.float32),
        pltpu.VMEM((RPT, BATCH), jnp.float32),
        pltpu.VMEM((RPT, BATCH), jnp.float32),
        pltpu.VMEM((RPT, BATCH), jnp.float32),
        pltpu.VMEM_SHARED((NGRP, DPC, BATCH), jnp.float32),
        pltpu.SemaphoreType.DMA,
        pltpu.SemaphoreType.DMA,
    ],
)(_sc_body)


@jax.jit
def kernel(input_one, input_two, weight):
    i1 = input_one.astype(jnp.int32)
    i2 = input_two.astype(jnp.int32)
    return _two_hot_sc(i1, i2, weight.T).T


# speedup vs baseline: 1.0956x; 1.0956x over previous
import functools

import jax
import jax.numpy as jnp
from jax import lax
from jax.experimental import pallas as pl
from jax.experimental.pallas import tpu as pltpu
from jax.experimental.pallas import tpu_sc as plsc

NUM_EMB = 100000
DIM = 64
BATCH = 1024

NUM_CORES = 2
NUM_SUBCORES = 16
L = 16
NCHUNK = 8
CW = 12544
CW_LAST = NUM_EMB - (NCHUNK - 1) * CW
CW_LAST_DMA = ((CW_LAST + 127) // 128) * 128
DPC = DIM // NUM_CORES
NGRP = DPC // 8
ROWS_PER_TILE = DPC // NUM_SUBCORES


def _sc_body(i1_hbm, i2_hbm, wt_hbm, out_hbm,
             i1_v, i2_v, scale_v, slab_v, part_v,
             fb0, fb1, fb2, fb3, out_v, shared, sem):
    s = lax.axis_index("s")
    c = lax.axis_index("c")
    glocal = s % NGRP
    owner = s // NGRP
    dbase = DPC * c + 8 * glocal

    pltpu.sync_copy(i1_hbm.at[pl.ds(0, BATCH)], i1_v)
    pltpu.sync_copy(i2_hbm.at[pl.ds(0, BATCH)], i2_v)

    iota = lax.iota(jnp.int32, L)

    def _prep(t, carry):
        a = i1_v[pl.ds(t * L, L)]
        b = i2_v[pl.ds(t * L, L)]
        scale_v[pl.ds(t * L, L)] = jnp.where(
            a != b, jnp.float32(1.0), jnp.float32(0.0))
        z = jnp.zeros((L,), jnp.float32)
        for dl in range(8):
            part_v[dl, pl.ds(t * L, L)] = z
        return carry

    for r in range(2):
        q = owner + NGRP * r
        cbase = pl.multiple_of(q * CW, 128)
        w = jnp.where(q == NCHUNK - 1, CW_LAST, CW).astype(jnp.int32)

        if r == 0:
            cp = pltpu.async_copy(
                wt_hbm.at[pl.ds(dbase, 8), pl.ds(cbase, CW)], slab_v, sem)
            lax.fori_loop(0, BATCH // L, _prep, 0)
            cp.wait()
        else:
            @pl.when(q == NCHUNK - 1)
            def _():
                pltpu.sync_copy(
                    wt_hbm.at[pl.ds(dbase, 8), pl.ds(cbase, CW_LAST_DMA)],
                    slab_v.at[:, pl.ds(0, CW_LAST_DMA)])

            @pl.when(q != NCHUNK - 1)
            def _():
                pltpu.sync_copy(
                    wt_hbm.at[pl.ds(dbase, 8), pl.ds(cbase, CW)], slab_v)

        def _accum(t, carry, idx_ref, scaled):
            idx = idx_ref[pl.ds(t * L, L)]
            local = idx - cbase
            m = (local >= 0) & (local < w)
            bvec = t * L + iota
            if scaled:
                scl = scale_v[pl.ds(t * L, L)]
            for dl in range(8):
                dsplat = jnp.full((L,), dl, jnp.int32)
                v = plsc.load_gather(slab_v, [dsplat, local], mask=m)
                if scaled:
                    v = v * scl
                plsc.addupdate_scatter(part_v, [dsplat, bvec], v, mask=m)
            return carry

        lax.fori_loop(0, BATCH // L,
                      functools.partial(_accum, idx_ref=i1_v, scaled=False), 0)
        lax.fori_loop(0, BATCH // L,
                      functools.partial(_accum, idx_ref=i2_v, scaled=True), 0)

    pltpu.sync_copy(part_v, shared.at[owner, pl.ds(8 * glocal, 8), :])
    plsc.subcore_barrier()

    rbase = ROWS_PER_TILE * s
    pltpu.sync_copy(shared.at[0, pl.ds(rbase, ROWS_PER_TILE), :], fb0)
    pltpu.sync_copy(shared.at[1, pl.ds(rbase, ROWS_PER_TILE), :], fb1)
    pltpu.sync_copy(shared.at[2, pl.ds(rbase, ROWS_PER_TILE), :], fb2)
    pltpu.sync_copy(shared.at[3, pl.ds(rbase, ROWS_PER_TILE), :], fb3)

    def _reduce(t, carry):
        for row in range(ROWS_PER_TILE):
            sl = pl.ds(t * L, L)
            out_v[row, sl] = ((fb0[row, sl] + fb1[row, sl])
                              + (fb2[row, sl] + fb3[row, sl]))
        return carry
    lax.fori_loop(0, BATCH // L, _reduce, 0)

    pltpu.sync_copy(out_v, out_hbm.at[pl.ds(DPC * c + rbase, ROWS_PER_TILE), :])


_two_hot_sc = functools.partial(
    pl.kernel,
    out_type=jax.ShapeDtypeStruct((DIM, BATCH), jnp.float32),
    mesh=plsc.VectorSubcoreMesh(core_axis_name="c", subcore_axis_name="s"),
    compiler_params=pltpu.CompilerParams(needs_layout_passes=False),
    scratch_types=[
        pltpu.VMEM((BATCH,), jnp.int32),
        pltpu.VMEM((BATCH,), jnp.int32),
        pltpu.VMEM((BATCH,), jnp.float32),
        pltpu.VMEM((8, CW), jnp.float32),
        pltpu.VMEM((8, BATCH), jnp.float32),
        pltpu.VMEM((ROWS_PER_TILE, BATCH), jnp.float32),
        pltpu.VMEM((ROWS_PER_TILE, BATCH), jnp.float32),
        pltpu.VMEM((ROWS_PER_TILE, BATCH), jnp.float32),
        pltpu.VMEM((ROWS_PER_TILE, BATCH), jnp.float32),
        pltpu.VMEM((ROWS_PER_TILE, BATCH), jnp.float32),
        pltpu.VMEM_SHARED((NGRP, DPC, BATCH), jnp.float32),
        pltpu.SemaphoreType.DMA,
    ],
)(_sc_body)


@jax.jit
def kernel(input_one, input_two, weight):
    i1 = input_one.astype(jnp.int32)
    i2 = input_two.astype(jnp.int32)
    return _two_hot_sc(i1, i2, weight.T).T


# speedup vs baseline: 1.2925x; 1.1797x over previous
import functools

import jax
import jax.numpy as jnp
from jax import lax
from jax.experimental import pallas as pl
from jax.experimental.pallas import tpu as pltpu
from jax.experimental.pallas import tpu_sc as plsc

NUM_EMB = 100000
DIM = 64
BATCH = 1024

NUM_CORES = 2
NUM_SUBCORES = 16
L = 16
NCHUNK = 8
CW = 12544
CW_LAST = NUM_EMB - (NCHUNK - 1) * CW
CW_LAST_DMA = ((CW_LAST + 127) // 128) * 128
DPC = DIM // NUM_CORES
NGRP = DPC // 8
ROWS_PER_TILE = DPC // NUM_SUBCORES


def _sc_body(i1_hbm, i2_hbm, wt_hbm, out_hbm,
             i1_v, i2_v, scale_v, slab_v, part_v,
             fb0, fb1, fb2, fb3, out_v, shared, sem):
    s = lax.axis_index("s")
    c = lax.axis_index("c")
    glocal = s % NGRP
    owner = s // NGRP
    dbase = DPC * c + 8 * glocal

    pltpu.sync_copy(i1_hbm.at[pl.ds(0, BATCH)], i1_v)
    pltpu.sync_copy(i2_hbm.at[pl.ds(0, BATCH)], i2_v)

    iota = lax.iota(jnp.int32, L)

    def _prep(t, carry):
        a = i1_v[pl.ds(t * L, L)]
        b = i2_v[pl.ds(t * L, L)]
        scale_v[pl.ds(t * L, L)] = jnp.where(
            a != b, jnp.float32(1.0), jnp.float32(0.0))
        z = jnp.zeros((L,), jnp.float32)
        for dl in range(8):
            part_v[dl, pl.ds(t * L, L)] = z
        return carry

    for r in range(2):
        q = owner + NGRP * r
        cbase = pl.multiple_of(q * CW, 128)
        w = jnp.where(q == NCHUNK - 1, CW_LAST, CW).astype(jnp.int32)

        if r == 0:
            cp = pltpu.async_copy(
                wt_hbm.at[pl.ds(dbase, 8), pl.ds(cbase, CW)], slab_v, sem)
            lax.fori_loop(0, BATCH // L, _prep, 0)
            cp.wait()
        else:
            @pl.when(q == NCHUNK - 1)
            def _():
                pltpu.sync_copy(
                    wt_hbm.at[pl.ds(dbase, 8), pl.ds(cbase, CW_LAST_DMA)],
                    slab_v.at[:, pl.ds(0, CW_LAST_DMA)])

            @pl.when(q != NCHUNK - 1)
            def _():
                pltpu.sync_copy(
                    wt_hbm.at[pl.ds(dbase, 8), pl.ds(cbase, CW)], slab_v)

        def _make_accum(idx_ref, scaled):
            def _accum(t):
                idx = idx_ref[pl.ds(t * L, L)]
                local = idx - cbase
                m = (local >= 0) & (local < w)
                bvec = t * L + iota
                scl = scale_v[pl.ds(t * L, L)] if scaled else None
                for dl in range(8):
                    dsplat = jnp.full((L,), dl, jnp.int32)
                    v = plsc.load_gather(slab_v, [dsplat, local], mask=m)
                    if scaled:
                        v = v * scl
                    plsc.addupdate_scatter(part_v, [dsplat, bvec], v, mask=m)
            return _accum

        plsc.parallel_loop(0, BATCH // L, unroll=2)(
            _make_accum(i1_v, False))
        plsc.parallel_loop(0, BATCH // L, unroll=2)(
            _make_accum(i2_v, True))

    pltpu.sync_copy(part_v, shared.at[owner, pl.ds(8 * glocal, 8), :])
    plsc.subcore_barrier()

    rbase = ROWS_PER_TILE * s
    pltpu.sync_copy(shared.at[0, pl.ds(rbase, ROWS_PER_TILE), :], fb0)
    pltpu.sync_copy(shared.at[1, pl.ds(rbase, ROWS_PER_TILE), :], fb1)
    pltpu.sync_copy(shared.at[2, pl.ds(rbase, ROWS_PER_TILE), :], fb2)
    pltpu.sync_copy(shared.at[3, pl.ds(rbase, ROWS_PER_TILE), :], fb3)

    def _reduce(t, carry):
        for row in range(ROWS_PER_TILE):
            sl = pl.ds(t * L, L)
            out_v[row, sl] = ((fb0[row, sl] + fb1[row, sl])
                              + (fb2[row, sl] + fb3[row, sl]))
        return carry
    lax.fori_loop(0, BATCH // L, _reduce, 0)

    pltpu.sync_copy(out_v, out_hbm.at[pl.ds(DPC * c + rbase, ROWS_PER_TILE), :])


_two_hot_sc = functools.partial(
    pl.kernel,
    out_type=jax.ShapeDtypeStruct((DIM, BATCH), jnp.float32),
    mesh=plsc.VectorSubcoreMesh(core_axis_name="c", subcore_axis_name="s"),
    compiler_params=pltpu.CompilerParams(needs_layout_passes=False),
    scratch_types=[
        pltpu.VMEM((BATCH,), jnp.int32),
        pltpu.VMEM((BATCH,), jnp.int32),
        pltpu.VMEM((BATCH,), jnp.float32),
        pltpu.VMEM((8, CW), jnp.float32),
        pltpu.VMEM((8, BATCH), jnp.float32),
        pltpu.VMEM((ROWS_PER_TILE, BATCH), jnp.float32),
        pltpu.VMEM((ROWS_PER_TILE, BATCH), jnp.float32),
        pltpu.VMEM((ROWS_PER_TILE, BATCH), jnp.float32),
        pltpu.VMEM((ROWS_PER_TILE, BATCH), jnp.float32),
        pltpu.VMEM((ROWS_PER_TILE, BATCH), jnp.float32),
        pltpu.VMEM_SHARED((NGRP, DPC, BATCH), jnp.float32),
        pltpu.SemaphoreType.DMA,
    ],
)(_sc_body)


@jax.jit
def kernel(input_one, input_two, weight):
    i1 = input_one.astype(jnp.int32)
    i2 = input_two.astype(jnp.int32)
    return _two_hot_sc(i1, i2, weight.T).T


# speedup vs baseline: 1.2928x; 1.0002x over previous
import functools

import jax
import jax.numpy as jnp
from jax import lax
from jax.experimental import pallas as pl
from jax.experimental.pallas import tpu as pltpu
from jax.experimental.pallas import tpu_sc as plsc

NUM_EMB = 100000
DIM = 64
BATCH = 1024

NUM_CORES = 2
NUM_SUBCORES = 16
L = 16
NCHUNK = 8
CW = 12544
CW_LAST = NUM_EMB - (NCHUNK - 1) * CW
CW_LAST_DMA = ((CW_LAST + 127) // 128) * 128
DPC = DIM // NUM_CORES
NGRP = DPC // 8
ROWS_PER_TILE = DPC // NUM_SUBCORES


def _sc_body(i1_hbm, i2_hbm, wt_hbm, out_hbm,
             i1_v, i2_v, scale_v, slab_v, part_v,
             fb0, fb1, fb2, fb3, out_v, shared, sem):
    s = lax.axis_index("s")
    c = lax.axis_index("c")
    glocal = s % NGRP
    owner = s // NGRP
    dbase = DPC * c + 8 * glocal

    pltpu.sync_copy(i1_hbm.at[pl.ds(0, BATCH)], i1_v)
    pltpu.sync_copy(i2_hbm.at[pl.ds(0, BATCH)], i2_v)

    iota = lax.iota(jnp.int32, L)

    def _prep(t):
        a = i1_v[pl.ds(t * L, L)]
        b = i2_v[pl.ds(t * L, L)]
        scale_v[pl.ds(t * L, L)] = jnp.where(
            a != b, jnp.float32(1.0), jnp.float32(0.0))
        z = jnp.zeros((L,), jnp.float32)
        for dl in range(8):
            part_v[dl, pl.ds(t * L, L)] = z

    for r in range(2):
        q = owner + NGRP * r
        cbase = pl.multiple_of(q * CW, 128)
        w = jnp.where(q == NCHUNK - 1, CW_LAST, CW).astype(jnp.int32)

        if r == 0:
            cp = pltpu.async_copy(
                wt_hbm.at[pl.ds(dbase, 8), pl.ds(cbase, CW)], slab_v, sem)
            plsc.parallel_loop(0, BATCH // L, unroll=2)(_prep)
            cp.wait()
        else:
            @pl.when(q == NCHUNK - 1)
            def _():
                pltpu.sync_copy(
                    wt_hbm.at[pl.ds(dbase, 8), pl.ds(cbase, CW_LAST_DMA)],
                    slab_v.at[:, pl.ds(0, CW_LAST_DMA)])

            @pl.when(q != NCHUNK - 1)
            def _():
                pltpu.sync_copy(
                    wt_hbm.at[pl.ds(dbase, 8), pl.ds(cbase, CW)], slab_v)

        def _make_accum(idx_ref, scaled):
            def _accum(t):
                idx = idx_ref[pl.ds(t * L, L)]
                local = idx - cbase
                m = (local >= 0) & (local < w)
                bvec = t * L + iota
                scl = scale_v[pl.ds(t * L, L)] if scaled else None
                for dl in range(8):
                    dsplat = jnp.full((L,), dl, jnp.int32)
                    v = plsc.load_gather(slab_v, [dsplat, local], mask=m)
                    if scaled:
                        v = v * scl
                    plsc.addupdate_scatter(part_v, [dsplat, bvec], v, mask=m)
            return _accum

        plsc.parallel_loop(0, BATCH // L, unroll=2)(
            _make_accum(i1_v, False))
        plsc.parallel_loop(0, BATCH // L, unroll=2)(
            _make_accum(i2_v, True))

    pltpu.sync_copy(part_v, shared.at[owner, pl.ds(8 * glocal, 8), :])
    plsc.subcore_barrier()

    rbase = ROWS_PER_TILE * s
    pltpu.sync_copy(shared.at[0, pl.ds(rbase, ROWS_PER_TILE), :], fb0)
    pltpu.sync_copy(shared.at[1, pl.ds(rbase, ROWS_PER_TILE), :], fb1)
    pltpu.sync_copy(shared.at[2, pl.ds(rbase, ROWS_PER_TILE), :], fb2)
    pltpu.sync_copy(shared.at[3, pl.ds(rbase, ROWS_PER_TILE), :], fb3)

    def _reduce(t):
        for row in range(ROWS_PER_TILE):
            sl = pl.ds(t * L, L)
            out_v[row, sl] = ((fb0[row, sl] + fb1[row, sl])
                              + (fb2[row, sl] + fb3[row, sl]))
    plsc.parallel_loop(0, BATCH // L, unroll=2)(_reduce)

    pltpu.sync_copy(out_v, out_hbm.at[pl.ds(DPC * c + rbase, ROWS_PER_TILE), :])


_two_hot_sc = functools.partial(
    pl.kernel,
    out_type=jax.ShapeDtypeStruct((DIM, BATCH), jnp.float32),
    mesh=plsc.VectorSubcoreMesh(core_axis_name="c", subcore_axis_name="s"),
    compiler_params=pltpu.CompilerParams(needs_layout_passes=False),
    scratch_types=[
        pltpu.VMEM((BATCH,), jnp.int32),
        pltpu.VMEM((BATCH,), jnp.int32),
        pltpu.VMEM((BATCH,), jnp.float32),
        pltpu.VMEM((8, CW), jnp.float32),
        pltpu.VMEM((8, BATCH), jnp.float32),
        pltpu.VMEM((ROWS_PER_TILE, BATCH), jnp.float32),
        pltpu.VMEM((ROWS_PER_TILE, BATCH), jnp.float32),
        pltpu.VMEM((ROWS_PER_TILE, BATCH), jnp.float32),
        pltpu.VMEM((ROWS_PER_TILE, BATCH), jnp.float32),
        pltpu.VMEM((ROWS_PER_TILE, BATCH), jnp.float32),
        pltpu.VMEM_SHARED((NGRP, DPC, BATCH), jnp.float32),
        pltpu.SemaphoreType.DMA,
    ],
)(_sc_body)


@jax.jit
def kernel(input_one, input_two, weight):
    i1 = input_one.astype(jnp.int32)
    i2 = input_two.astype(jnp.int32)
    return _two_hot_sc(i1, i2, weight.T).T
